# baseline (speedup 1.0000x reference)
import functools

import jax
import jax.numpy as jnp
from jax import lax
from jax.experimental import pallas as pl
from jax.experimental.pallas import tpu as pltpu
from jax.experimental.pallas import tpu_sc as plsc

B = 1024
L = 50
K = 32
PHI = 3
VOCAB = 1000000
N = B * L
NC = 2
NS = 16
NW = NC * NS
BSUB = B // NW

SCV = 16384


def _make_split(width):
  def body(in_ref, *out_refs):
    x = in_ref[...]
    for k in range(width):
      out_refs[k][...] = x[k]

  grid = (VOCAB + SCV - 1) // SCV
  out = jax.ShapeDtypeStruct((VOCAB,), jnp.float32)
  return pl.pallas_call(
      body,
      grid=(grid,),
      in_specs=[pl.BlockSpec((width, SCV), lambda c: (0, c))],
      out_specs=[pl.BlockSpec((SCV,), lambda c: (c,))] * width,
      out_shape=[out] * width,
  )


def _sc_ls_body(tok_hbm, *refs):
  planes = refs[:K]
  ls_out = refs[K]
  idx_v, ls_v, sem = refs[K + 1:]
  wid = lax.axis_index("s") * NC + lax.axis_index("c")
  pltpu.sync_copy(tok_hbm.at[:, pl.ds(wid * BSUB, BSUB)], idx_v)

  def per_l(l, carry):
    for k in range(K):
      pltpu.async_copy(planes[k].at[idx_v.at[l]], ls_v.at[l, k], sem)
    return carry

  lax.fori_loop(0, L, per_l, 0)
  pltpu.make_async_copy(ls_out.at[:, :, wid], ls_v, sem).wait()
  pltpu.sync_copy(ls_v, ls_out.at[:, :, wid])


def _make_sc_ls():
  mesh = plsc.VectorSubcoreMesh(core_axis_name="c", subcore_axis_name="s")
  return pl.kernel(
      _sc_ls_body,
      mesh=mesh,
      out_type=jax.ShapeDtypeStruct((L, K, NW, BSUB), jnp.float32),
      scratch_types=[
          pltpu.VMEM((L, BSUB), jnp.int32),
          pltpu.VMEM((L, K, BSUB), jnp.float32),
          pltpu.SemaphoreType.DMA,
      ],
      compiler_params=pltpu.CompilerParams(use_tc_tiling_on_sc=False),
  )


def _sc_mu_phi_body(tok_hbm, *refs):
  mu_planes = refs[:K]
  phi_planes = refs[K:K + PHI]
  mu_out, phi_out = refs[K + PHI:K + PHI + 2]
  idx_v, mu_v, phi_v, sem = refs[K + PHI + 2:]
  wid = lax.axis_index("s") * NC + lax.axis_index("c")
  pltpu.sync_copy(tok_hbm.at[:, pl.ds(wid * BSUB, BSUB)], idx_v)

  def per_l(l, carry):
    for k in range(K):
      pltpu.async_copy(mu_planes[k].at[idx_v.at[l]], mu_v.at[l, k], sem)
    for p in range(PHI):
      pltpu.async_copy(phi_planes[p].at[idx_v.at[l]], phi_v.at[p, l], sem)
    return carry

  lax.fori_loop(0, L, per_l, 0)
  pltpu.make_async_copy(mu_out.at[:, :, wid], mu_v, sem).wait()
  pltpu.make_async_copy(phi_out.at[:, :, wid], phi_v, sem).wait()
  pltpu.sync_copy(mu_v, mu_out.at[:, :, wid])
  pltpu.sync_copy(phi_v, phi_out.at[:, :, wid])


def _make_sc_mu_phi():
  mesh = plsc.VectorSubcoreMesh(core_axis_name="c", subcore_axis_name="s")
  return pl.kernel(
      _sc_mu_phi_body,
      mesh=mesh,
      out_type=[
          jax.ShapeDtypeStruct((L, K, NW, BSUB), jnp.float32),
          jax.ShapeDtypeStruct((PHI, L, NW, BSUB), jnp.float32),
      ],
      scratch_types=[
          pltpu.VMEM((L, BSUB), jnp.int32),
          pltpu.VMEM((L, K, BSUB), jnp.float32),
          pltpu.VMEM((PHI, L, BSUB), jnp.float32),
          pltpu.SemaphoreType.DMA,
      ],
      compiler_params=pltpu.CompilerParams(use_tc_tiling_on_sc=False),
  )


def _expand_body(ls_ref, out_ref):
  sd = jnp.exp(ls_ref[...])
  i = lax.broadcasted_iota(jnp.int32, (1, K, K, B), 1)
  j = lax.broadcasted_iota(jnp.int32, (1, K, K, B), 2)
  out_ref[...] = jnp.where(i == j, sd[None, :, None, :], 0.0)


def _expand(ls2):
  return pl.pallas_call(
      _expand_body,
      grid=(L,),
      in_specs=[pl.BlockSpec((K, B), lambda l: (l, 0))],
      out_specs=pl.BlockSpec((1, K, K, B), lambda l: (l, 0, 0, 0)),
      out_shape=jax.ShapeDtypeStruct((L, K, K, B), jnp.float32),
  )(ls2)


def kernel(token_ids, mu_table, log_sigma_diag, phi_table):
  tok_t = token_ids.T

  ls_planes = _make_split(K)(log_sigma_diag.T)
  ls4 = _make_sc_ls()(tok_t, *ls_planes)
  sigma_likb = _expand(ls4.reshape(L * K, B))

  mu_planes = _make_split(K)(mu_table.T)
  phi_planes = _make_split(PHI)(phi_table.T)
  mu4, phi4 = _make_sc_mu_phi()(tok_t, *mu_planes, *phi_planes)

  return (jnp.transpose(mu4.reshape(L, K, B), (2, 0, 1)),
          jnp.transpose(sigma_likb, (3, 0, 1, 2)),
          jnp.transpose(phi4.reshape(PHI, L, B), (2, 1, 0)))


# speedup vs baseline: 1.0995x; 1.0995x over previous
import functools

import jax
import jax.numpy as jnp
from jax import lax
from jax.experimental import pallas as pl
from jax.experimental.pallas import tpu as pltpu
from jax.experimental.pallas import tpu_sc as plsc

B = 1024
L = 50
K = 32
PHI = 3
VOCAB = 1000000
N = B * L
NC = 2
NS = 16
NW = NC * NS
BSUB = B // NW

SCV = 32768


def _make_split(width):
  def body(in_ref, *out_refs):
    x = in_ref[...]
    for k in range(width):
      out_refs[k][...] = x[k]

  grid = (VOCAB + SCV - 1) // SCV
  out = jax.ShapeDtypeStruct((VOCAB,), jnp.float32)
  return pl.pallas_call(
      body,
      grid=(grid,),
      in_specs=[pl.BlockSpec((width, SCV), lambda c: (0, c))],
      out_specs=[pl.BlockSpec((SCV,), lambda c: (c,))] * width,
      out_shape=[out] * width,
  )


def _sc_ls_body(tok_hbm, *refs):
  planes = refs[:K]
  ls_out = refs[K]
  idx_v, ls_v, sem = refs[K + 1:]
  wid = lax.axis_index("s") * NC + lax.axis_index("c")
  pltpu.sync_copy(tok_hbm.at[:, pl.ds(wid * BSUB, BSUB)], idx_v)

  def per_l(l, carry):
    for k in range(K):
      pltpu.async_copy(planes[k].at[idx_v.at[l]], ls_v.at[l, k], sem)
    return carry

  lax.fori_loop(0, L, per_l, 0)
  pltpu.make_async_copy(ls_out.at[:, :, wid], ls_v, sem).wait()
  pltpu.sync_copy(ls_v, ls_out.at[:, :, wid])


def _make_sc_ls():
  mesh = plsc.VectorSubcoreMesh(core_axis_name="c", subcore_axis_name="s")
  return pl.kernel(
      _sc_ls_body,
      mesh=mesh,
      out_type=jax.ShapeDtypeStruct((L, K, NW, BSUB), jnp.float32),
      scratch_types=[
          pltpu.VMEM((L, BSUB), jnp.int32),
          pltpu.VMEM((L, K, BSUB), jnp.float32),
          pltpu.SemaphoreType.DMA,
      ],
      compiler_params=pltpu.CompilerParams(use_tc_tiling_on_sc=False),
  )


def _sc_mu_phi_body(tok_hbm, *refs):
  mu_planes = refs[:K]
  phi_planes = refs[K:K + PHI]
  mu_out, phi_out = refs[K + PHI:K + PHI + 2]
  idx_v, mu_v, phi_v, sem = refs[K + PHI + 2:]
  wid = lax.axis_index("s") * NC + lax.axis_index("c")
  pltpu.sync_copy(tok_hbm.at[:, pl.ds(wid * BSUB, BSUB)], idx_v)

  def per_l(l, carry):
    for k in range(K):
      pltpu.async_copy(mu_planes[k].at[idx_v.at[l]], mu_v.at[l, k], sem)
    for p in range(PHI):
      pltpu.async_copy(phi_planes[p].at[idx_v.at[l]], phi_v.at[p, l], sem)
    return carry

  lax.fori_loop(0, L, per_l, 0)
  pltpu.make_async_copy(mu_out.at[:, :, wid], mu_v, sem).wait()
  pltpu.make_async_copy(phi_out.at[:, :, wid], phi_v, sem).wait()
  pltpu.sync_copy(mu_v, mu_out.at[:, :, wid])
  pltpu.sync_copy(phi_v, phi_out.at[:, :, wid])


def _make_sc_mu_phi():
  mesh = plsc.VectorSubcoreMesh(core_axis_name="c", subcore_axis_name="s")
  return pl.kernel(
      _sc_mu_phi_body,
      mesh=mesh,
      out_type=[
          jax.ShapeDtypeStruct((L, K, NW, BSUB), jnp.float32),
          jax.ShapeDtypeStruct((PHI, L, NW, BSUB), jnp.float32),
      ],
      scratch_types=[
          pltpu.VMEM((L, BSUB), jnp.int32),
          pltpu.VMEM((L, K, BSUB), jnp.float32),
          pltpu.VMEM((PHI, L, BSUB), jnp.float32),
          pltpu.SemaphoreType.DMA,
      ],
      compiler_params=pltpu.CompilerParams(use_tc_tiling_on_sc=False),
  )


def _expand_body(ls_ref, out_ref):
  sd = jnp.exp(ls_ref[...])
  i = lax.broadcasted_iota(jnp.int32, (1, K, K, B), 1)
  j = lax.broadcasted_iota(jnp.int32, (1, K, K, B), 2)
  out_ref[...] = jnp.where(i == j, sd[None, :, None, :], 0.0)


def _expand(ls2):
  return pl.pallas_call(
      _expand_body,
      grid=(L,),
      in_specs=[pl.BlockSpec((K, B), lambda l: (l, 0))],
      out_specs=pl.BlockSpec((1, K, K, B), lambda l: (l, 0, 0, 0)),
      out_shape=jax.ShapeDtypeStruct((L, K, K, B), jnp.float32),
  )(ls2)


def kernel(token_ids, mu_table, log_sigma_diag, phi_table):
  tok_t = token_ids.T

  ls_planes = _make_split(K)(log_sigma_diag.T)
  ls4 = _make_sc_ls()(tok_t, *ls_planes)
  sigma_likb = _expand(ls4.reshape(L * K, B))

  mu_planes = _make_split(K)(mu_table.T)
  phi_planes = _make_split(PHI)(phi_table.T)
  mu4, phi4 = _make_sc_mu_phi()(tok_t, *mu_planes, *phi_planes)

  return (jnp.transpose(mu4.reshape(L, K, B), (2, 0, 1)),
          jnp.transpose(sigma_likb, (3, 0, 1, 2)),
          jnp.transpose(phi4.reshape(PHI, L, B), (2, 1, 0)))


# speedup vs baseline: 1.1281x; 1.0260x over previous
import functools

import jax
import jax.numpy as jnp
from jax import lax
from jax.experimental import pallas as pl
from jax.experimental.pallas import tpu as pltpu
from jax.experimental.pallas import tpu_sc as plsc

B = 1024
L = 50
K = 32
PHI = 3
VOCAB = 1000000
N = B * L
NC = 2
NS = 16
NW = NC * NS
BSUB = B // NW

SCV = 65536


def _make_split(width):
  def body(in_ref, *out_refs):
    x = in_ref[...]
    for k in range(width):
      out_refs[k][...] = x[k]

  grid = (VOCAB + SCV - 1) // SCV
  out = jax.ShapeDtypeStruct((VOCAB,), jnp.float32)
  return pl.pallas_call(
      body,
      grid=(grid,),
      in_specs=[pl.BlockSpec((width, SCV), lambda c: (0, c))],
      out_specs=[pl.BlockSpec((SCV,), lambda c: (c,))] * width,
      out_shape=[out] * width,
  )


def _sc_ls_body(tok_hbm, *refs):
  planes = refs[:K]
  ls_out = refs[K]
  idx_v, ls_v, sem = refs[K + 1:]
  wid = lax.axis_index("s") * NC + lax.axis_index("c")
  pltpu.sync_copy(tok_hbm.at[:, pl.ds(wid * BSUB, BSUB)], idx_v)

  def per_l(l, carry):
    for k in range(K):
      pltpu.async_copy(planes[k].at[idx_v.at[l]], ls_v.at[l, k], sem)
    return carry

  lax.fori_loop(0, L, per_l, 0)
  pltpu.make_async_copy(ls_out.at[:, :, wid], ls_v, sem).wait()
  pltpu.sync_copy(ls_v, ls_out.at[:, :, wid])


def _make_sc_ls():
  mesh = plsc.VectorSubcoreMesh(core_axis_name="c", subcore_axis_name="s")
  return pl.kernel(
      _sc_ls_body,
      mesh=mesh,
      out_type=jax.ShapeDtypeStruct((L, K, NW, BSUB), jnp.float32),
      scratch_types=[
          pltpu.VMEM((L, BSUB), jnp.int32),
          pltpu.VMEM((L, K, BSUB), jnp.float32),
          pltpu.SemaphoreType.DMA,
      ],
      compiler_params=pltpu.CompilerParams(use_tc_tiling_on_sc=False),
  )


def _sc_mu_phi_body(tok_hbm, *refs):
  mu_planes = refs[:K]
  phi_planes = refs[K:K + PHI]
  mu_out, phi_out = refs[K + PHI:K + PHI + 2]
  idx_v, mu_v, phi_v, sem = refs[K + PHI + 2:]
  wid = lax.axis_index("s") * NC + lax.axis_index("c")
  pltpu.sync_copy(tok_hbm.at[:, pl.ds(wid * BSUB, BSUB)], idx_v)

  def per_l(l, carry):
    for k in range(K):
      pltpu.async_copy(mu_planes[k].at[idx_v.at[l]], mu_v.at[l, k], sem)
    for p in range(PHI):
      pltpu.async_copy(phi_planes[p].at[idx_v.at[l]], phi_v.at[p, l], sem)
    return carry

  lax.fori_loop(0, L, per_l, 0)
  pltpu.make_async_copy(mu_out.at[:, :, wid], mu_v, sem).wait()
  pltpu.make_async_copy(phi_out.at[:, :, wid], phi_v, sem).wait()
  pltpu.sync_copy(mu_v, mu_out.at[:, :, wid])
  pltpu.sync_copy(phi_v, phi_out.at[:, :, wid])


def _make_sc_mu_phi():
  mesh = plsc.VectorSubcoreMesh(core_axis_name="c", subcore_axis_name="s")
  return pl.kernel(
      _sc_mu_phi_body,
      mesh=mesh,
      out_type=[
          jax.ShapeDtypeStruct((L, K, NW, BSUB), jnp.float32),
          jax.ShapeDtypeStruct((PHI, L, NW, BSUB), jnp.float32),
      ],
      scratch_types=[
          pltpu.VMEM((L, BSUB), jnp.int32),
          pltpu.VMEM((L, K, BSUB), jnp.float32),
          pltpu.VMEM((PHI, L, BSUB), jnp.float32),
          pltpu.SemaphoreType.DMA,
      ],
      compiler_params=pltpu.CompilerParams(use_tc_tiling_on_sc=False),
  )


def _expand_body(ls_ref, out_ref):
  sd = jnp.exp(ls_ref[...])
  i = lax.broadcasted_iota(jnp.int32, (1, K, K, B), 1)
  j = lax.broadcasted_iota(jnp.int32, (1, K, K, B), 2)
  out_ref[...] = jnp.where(i == j, sd[None, :, None, :], 0.0)


def _expand(ls2):
  return pl.pallas_call(
      _expand_body,
      grid=(L,),
      in_specs=[pl.BlockSpec((K, B), lambda l: (l, 0))],
      out_specs=pl.BlockSpec((1, K, K, B), lambda l: (l, 0, 0, 0)),
      out_shape=jax.ShapeDtypeStruct((L, K, K, B), jnp.float32),
  )(ls2)


def kernel(token_ids, mu_table, log_sigma_diag, phi_table):
  tok_t = token_ids.T

  ls_planes = _make_split(K)(log_sigma_diag.T)
  ls4 = _make_sc_ls()(tok_t, *ls_planes)
  sigma_likb = _expand(ls4.reshape(L * K, B))

  mu_planes = _make_split(K)(mu_table.T)
  phi_planes = _make_split(PHI)(phi_table.T)
  mu4, phi4 = _make_sc_mu_phi()(tok_t, *mu_planes, *phi_planes)

  return (jnp.transpose(mu4.reshape(L, K, B), (2, 0, 1)),
          jnp.transpose(sigma_likb, (3, 0, 1, 2)),
          jnp.transpose(phi4.reshape(PHI, L, B), (2, 1, 0)))
